# baseline (speedup 1.0000x reference)
import jax
import jax.numpy as jnp
import numpy as np
from jax import lax
from jax.experimental import pallas as pl
from jax.experimental.pallas import tpu as pltpu
from jax.experimental.pallas import tpu_sc as plsc

B, T, V, C = 64, 24, 24, 1001
NC, NS, L = 2, 16, 16
NW = NC * NS
BL = B // NW

TV = T * V
NCELL = BL * TV
RCH = 32
NCH = NCELL // RCH
W = V + 1
APB = 640
ADUM = 2 * APB
ASIZE = ADUM + 2 * L
CDUM = NCELL
CSIZE = CDUM + L
NEG = -1.0e30


def _build_tables():
    emap = np.zeros((NCH, 3, 2, L), np.int32)
    for ch in range(NCH):
        for s in range(3):
            for h in range(2):
                for j in range(L):
                    cc = ch * RCH + h * L + j
                    b, rem = divmod(cc, TV)
                    t, v = divmod(rem, V)
                    if s == 0:
                        emap[ch, s, h, j] = 48 + b * V + v
                    elif s == 1:
                        emap[ch, s, h, j] = b * T + t
                    else:
                        emap[ch, s, h, j] = 96 + cc
    emap = emap.reshape(-1)

    aidx_rows, sidx_rows, cidx_rows = [], [], []
    for d in range(1, T + V - 1):
        cells = [(b, t, d - t)
                 for b in range(BL)
                 for t in range(max(0, d - (V - 1)), min(T - 1, d) + 1)]
        for c0 in range(0, len(cells), L):
            chunk = cells[c0:c0 + L]
            ai = [ADUM + j for j in range(L)]
            si = [0] * L
            ci = [CDUM + j for j in range(L)]
            for j, (b, t, v) in enumerate(chunk):
                ai[j] = b * APB + (t + 1) * W + (v + 1)
                si[j] = b * TV + t * V + v
                ci[j] = si[j]
            aidx_rows.append(ai)
            sidx_rows.append(si)
            cidx_rows.append(ci)
    aidx = np.array(aidx_rows, np.int32).reshape(-1)
    sidx = np.array(sidx_rows, np.int32).reshape(-1)
    cidx = np.array(cidx_rows, np.int32).reshape(-1)
    ndp = len(aidx_rows)

    init_entries = []
    for b in range(BL):
        for vv in range(W):
            init_entries.append((b * APB + vv, NEG))
        for tt in range(1, W):
            init_entries.append((b * APB + tt * W, NEG))
        init_entries.append((b * APB + W + 1, 0.0))
    for j in range(2 * L):
        init_entries.append((ADUM + j, NEG))
    pad = 0
    while len(init_entries) % L:
        init_entries.append((APB - 16 + pad, NEG))
        pad += 1
    init_idx = np.array([e[0] for e in init_entries], np.int32)
    init_val = np.array([e[1] for e in init_entries], np.float32)

    cinit_idx = np.array([0, TV] + [CDUM + j for j in range(L - 2)], np.int32)
    cinit_val = np.array([0.0, 0.0] + [NEG] * (L - 2), np.float32)

    tabi = np.concatenate([emap, aidx, sidx, cidx, init_idx, cinit_idx])
    tabf = np.concatenate([init_val, cinit_val])
    offs = {}
    o = 0
    for name, a in (("emap", emap), ("aidx", aidx), ("sidx", sidx),
                    ("cidx", cidx), ("init_idx", init_idx),
                    ("cinit_idx", cinit_idx)):
        offs[name] = o
        o += a.size
    return tabi, tabf, offs, ndp, init_idx.size // L


_TABI, _TABF, _OFFS, _NDP, _NINIT = _build_tables()
_EMAP0 = _OFFS["emap"]
_AIDX0 = _OFFS["aidx"]
_SIDX0 = _OFFS["sidx"]
_CIDX0 = _OFFS["cidx"]
_INITI0 = _OFFS["init_idx"]
_CINITI0 = _OFFS["cinit_idx"]

_LN2 = 0.6931471805599453
_SQRT2 = 1.4142135623730951


def _log1to4(s):
    bits = plsc.bitcast(s, jnp.int32)
    e = (bits >> 23) - 127
    mant = plsc.bitcast((bits & 0x007FFFFF) | 0x3F800000, jnp.float32)
    big = mant > _SQRT2
    mant = jnp.where(big, mant * 0.5, mant)
    e = e + big.astype(jnp.int32)
    u = (mant - 1.0) / (mant + 1.0)
    u2 = u * u
    p = 2.0 * u * (1.0 + u2 * (1.0 / 3.0 + u2 * (0.2 + u2 * (1.0 / 7.0
                                                             + u2 * (1.0 / 9.0)))))
    return e.astype(jnp.float32) * _LN2 + p


def _body(scores_hbm, del_hbm, ins_hbm, sub_hbm, tabi_hbm, tabf_hbm,
          out_hbm,
          ids_v, tabi_v, tabf_v, dscores_v, alpha_v, compact_v,
          blk0, blk1, sem0, sem1):
    wid = lax.axis_index("s") * NC + lax.axis_index("c")
    b0 = wid * BL
    gr0 = b0 * TV

    pltpu.sync_copy(tabi_hbm, tabi_v)
    pltpu.sync_copy(tabf_hbm, tabf_v)
    pltpu.sync_copy(del_hbm.at[pl.ds(b0 * T, BL * T)], ids_v.at[pl.ds(0, 48)])
    pltpu.sync_copy(ins_hbm.at[pl.ds(b0 * V, BL * V)], ids_v.at[pl.ds(48, 48)])
    pltpu.sync_copy(sub_hbm.at[pl.ds(gr0, NCELL)], ids_v.at[pl.ds(96, NCELL)])

    pltpu.async_copy(scores_hbm.at[pl.ds(gr0, RCH), :], blk0, sem0)
    pltpu.async_copy(scores_hbm.at[pl.ds(gr0 + RCH, RCH), :], blk1, sem1)

    for k in range(_NINIT):
        idxv = tabi_v[pl.ds(_INITI0 + k * L, L)]
        valv = tabf_v[pl.ds(k * L, L)]
        plsc.store_scatter(alpha_v, [idxv], valv)
    plsc.store_scatter(compact_v, [tabi_v[pl.ds(_CINITI0, L)]],
                       tabf_v[pl.ds(_NINIT * L, L)])

    rows_lo = lax.iota(jnp.int32, L)
    rows_hi = rows_lo + L

    @pl.loop(0, NCH // 2)
    def _stream(p):
        for par, blk, sem in ((0, blk0, sem0), (1, blk1, sem1)):
            ch = p * 2 + par
            pltpu.make_async_copy(scores_hbm.at[pl.ds(0, RCH), :], blk,
                                  sem).wait()
            for s in range(3):
                for h, rowv in ((0, rows_lo), (1, rows_hi)):
                    moff = pl.multiple_of(
                        _EMAP0 + ch * (6 * L) + (s * 2 + h) * L, L)
                    cv = plsc.load_gather(ids_v, [tabi_v[pl.ds(moff, L)]])
                    val = plsc.load_gather(blk, [rowv, cv])
                    doff = pl.multiple_of(s * NCELL + ch * RCH + h * L, L)
                    dscores_v[pl.ds(doff, L)] = val

            @pl.when(p < NCH // 2 - 1)
            def _prefetch():
                nxt = pl.multiple_of(gr0 + (ch + 2) * RCH, RCH)
                pltpu.async_copy(scores_hbm.at[pl.ds(nxt, RCH), :], blk, sem)

    @pl.loop(0, _NDP)
    def _dp(i):
        off = pl.multiple_of(i * L, L)
        ai = tabi_v[pl.ds(_AIDX0 + off, L)]
        si = tabi_v[pl.ds(_SIDX0 + off, L)]
        ci = tabi_v[pl.ds(_CIDX0 + off, L)]
        insv = plsc.load_gather(dscores_v, [si])
        delv = plsc.load_gather(dscores_v, [si + NCELL])
        subv = plsc.load_gather(dscores_v, [si + 2 * NCELL])
        a_l = plsc.load_gather(alpha_v, [ai - 1])
        a_u = plsc.load_gather(alpha_v, [ai - W])
        a_d = plsc.load_gather(alpha_v, [ai - (W + 1)])
        x1 = insv + a_l
        x2 = delv + a_u
        x3 = subv + a_d
        m = jnp.maximum(x1, jnp.maximum(x2, x3))
        s = jnp.exp(x1 - m) + jnp.exp(x2 - m) + jnp.exp(x3 - m)
        r = m + _log1to4(s)
        plsc.store_scatter(alpha_v, [ai], r)
        plsc.store_scatter(compact_v, [ci], r)

    for b in range(BL):
        pltpu.sync_copy(compact_v.at[pl.ds(b * TV, TV)],
                        out_hbm.at[pl.ds((b0 + b) * TV, TV)])


@jax.jit
def _edit_dist_sc(scores2d, del_flat, ins_flat, sub_flat):
    mesh = plsc.VectorSubcoreMesh(core_axis_name="c", subcore_axis_name="s",
                                  num_cores=NC, num_subcores=NS)
    fn = pl.kernel(
        _body,
        out_type=jax.ShapeDtypeStruct((B * TV,), jnp.float32),
        mesh=mesh,
        compiler_params=pltpu.CompilerParams(needs_layout_passes=False),
        scratch_types=[
            pltpu.VMEM((96 + NCELL,), jnp.int32),
            pltpu.VMEM((_TABI.size,), jnp.int32),
            pltpu.VMEM((_TABF.size,), jnp.float32),
            pltpu.VMEM((3 * NCELL,), jnp.float32),
            pltpu.VMEM((ASIZE,), jnp.float32),
            pltpu.VMEM((CSIZE,), jnp.float32),
            pltpu.VMEM((RCH, C), jnp.float32),
            pltpu.VMEM((RCH, C), jnp.float32),
            pltpu.SemaphoreType.DMA,
            pltpu.SemaphoreType.DMA,
        ],
    )
    return fn(scores2d, del_flat, ins_flat, sub_flat,
              jnp.asarray(_TABI), jnp.asarray(_TABF))


def kernel(all_deletion_ids, all_insertion_ids, all_subs_ids, action_scores):
    out = _edit_dist_sc(
        action_scores.reshape(B * T * V, C),
        all_deletion_ids.reshape(-1).astype(jnp.int32),
        all_insertion_ids.reshape(-1).astype(jnp.int32),
        all_subs_ids.reshape(-1).astype(jnp.int32),
    )
    return out.reshape(B, T, V)


# speedup vs baseline: 1.0447x; 1.0447x over previous
import jax
import jax.numpy as jnp
import numpy as np
from jax import lax
from jax.experimental import pallas as pl
from jax.experimental.pallas import tpu as pltpu
from jax.experimental.pallas import tpu_sc as plsc

B, T, V, C = 64, 24, 24, 1001
NC, NS, L = 2, 16, 16
NW = NC * NS
BL = B // NW

TV = T * V
NCELL = BL * TV
RCH = 16
NP = TV // RCH
W = V + 1
APB = 640
ADUM = 2 * APB
ASIZE = ADUM + 2 * L
CDUM = NCELL
CSIZE = CDUM + L
NEG = -1.0e30


def _build_tables():
    emap = np.zeros((NCELL // L, 3, L), np.int32)
    for g in range(NCELL // L):
        for s in range(3):
            for j in range(L):
                cc = g * L + j
                b, rem = divmod(cc, TV)
                t, v = divmod(rem, V)
                if s == 0:
                    emap[g, s, j] = 48 + b * V + v
                elif s == 1:
                    emap[g, s, j] = b * T + t
                else:
                    emap[g, s, j] = 96 + cc
    emap = emap.reshape(-1)

    aidx_rows, sidx_rows, cidx_rows, diag_of_chunk = [], [], [], []
    for d in range(1, T + V - 1):
        cells = [(b, t, d - t)
                 for b in range(BL)
                 for t in range(max(0, d - (V - 1)), min(T - 1, d) + 1)]
        for c0 in range(0, len(cells), L):
            chunk = cells[c0:c0 + L]
            ai = [ADUM + j for j in range(L)]
            si = [0] * L
            ci = [CDUM + j for j in range(L)]
            for j, (b, t, v) in enumerate(chunk):
                ai[j] = b * APB + (t + 1) * W + (v + 1)
                si[j] = b * TV + t * V + v
                ci[j] = si[j]
            aidx_rows.append(ai)
            sidx_rows.append(si)
            cidx_rows.append(ci)
            diag_of_chunk.append(d)
    aidx = np.array(aidx_rows, np.int32).reshape(-1)
    sidx = np.array(sidx_rows, np.int32).reshape(-1)
    cidx = np.array(cidx_rows, np.int32).reshape(-1)
    ndp = len(aidx_rows)

    maxrow = np.zeros(T + V - 1, np.int64)
    for d in range(1, T + V - 1):
        maxrow[d] = max(t * V + (d - t)
                        for t in range(max(0, d - (V - 1)), min(T - 1, d) + 1))
    prefmax = np.maximum.accumulate(maxrow)
    dp_hi = []
    for p in range(NP):
        bound = RCH * (p + 1)
        dmax = 0
        for d in range(1, T + V - 1):
            if prefmax[d] < bound:
                dmax = d
            else:
                break
        hi = sum(1 for dd in diag_of_chunk if dd <= dmax)
        dp_hi.append(hi)
    assert dp_hi[-1] <= ndp
    lo = 0
    for p in range(NP):
        for k in range(lo, dp_hi[p]):
            for cc in sidx[k * L:(k + 1) * L]:
                assert cc % TV < RCH * (p + 1), (p, k, cc)
        lo = dp_hi[p]

    init_entries = []
    for b in range(BL):
        for vv in range(W):
            init_entries.append((b * APB + vv, NEG))
        for tt in range(1, W):
            init_entries.append((b * APB + tt * W, NEG))
        init_entries.append((b * APB + W + 1, 0.0))
    for j in range(2 * L):
        init_entries.append((ADUM + j, NEG))
    pad = 0
    while len(init_entries) % L:
        init_entries.append((APB - 16 + pad, NEG))
        pad += 1
    init_idx = np.array([e[0] for e in init_entries], np.int32)
    init_val = np.array([e[1] for e in init_entries], np.float32)

    cinit_idx = np.array([0, TV] + [CDUM + j for j in range(L - 2)], np.int32)
    cinit_val = np.array([0.0, 0.0] + [NEG] * (L - 2), np.float32)

    tabi = np.concatenate([emap, aidx, sidx, cidx, init_idx, cinit_idx])
    tabf = np.concatenate([init_val, cinit_val])
    offs = {}
    o = 0
    for name, a in (("emap", emap), ("aidx", aidx), ("sidx", sidx),
                    ("cidx", cidx), ("init_idx", init_idx),
                    ("cinit_idx", cinit_idx)):
        offs[name] = o
        o += a.size
    return tabi, tabf, offs, ndp, init_idx.size // L, dp_hi


_TABI, _TABF, _OFFS, _NDP, _NINIT, _DP_HI = _build_tables()
_EMAP0 = _OFFS["emap"]
_AIDX0 = _OFFS["aidx"]
_SIDX0 = _OFFS["sidx"]
_CIDX0 = _OFFS["cidx"]
_INITI0 = _OFFS["init_idx"]
_CINITI0 = _OFFS["cinit_idx"]

_LN2 = 0.6931471805599453
_SQRT2 = 1.4142135623730951


def _log1to4(s):
    bits = plsc.bitcast(s, jnp.int32)
    e = (bits >> 23) - 127
    mant = plsc.bitcast((bits & 0x007FFFFF) | 0x3F800000, jnp.float32)
    big = mant > _SQRT2
    mant = jnp.where(big, mant * 0.5, mant)
    e = e + big.astype(jnp.int32)
    u = (mant - 1.0) / (mant + 1.0)
    u2 = u * u
    p = 2.0 * u * (1.0 + u2 * (1.0 / 3.0 + u2 * (0.2 + u2 * (1.0 / 7.0
                                                             + u2 * (1.0 / 9.0)))))
    return e.astype(jnp.float32) * _LN2 + p


def _body(scores_hbm, del_hbm, ins_hbm, sub_hbm, tabi_hbm, tabf_hbm,
          out_hbm,
          ids_v, tabi_v, tabf_v, dscores_v, alpha_v, compact_v,
          blk00, blk01, blk10, blk11, sem00, sem01, sem10, sem11):
    wid = lax.axis_index("s") * NC + lax.axis_index("c")
    b0 = wid * BL
    gr0 = b0 * TV

    blks = ((blk00, blk01), (blk10, blk11))
    sems = ((sem00, sem01), (sem10, sem11))

    def fetch(b, q):
        pltpu.async_copy(
            scores_hbm.at[pl.ds(gr0 + b * TV + q * RCH, RCH), :],
            blks[b][q % 2], sems[b][q % 2])

    def wait(b, q):
        pltpu.make_async_copy(scores_hbm.at[pl.ds(0, RCH), :],
                              blks[b][q % 2], sems[b][q % 2]).wait()

    for b in range(BL):
        fetch(b, 0)
        fetch(b, 1)
    pltpu.sync_copy(tabi_hbm, tabi_v)
    pltpu.sync_copy(tabf_hbm, tabf_v)
    pltpu.sync_copy(del_hbm.at[pl.ds(b0 * T, BL * T)], ids_v.at[pl.ds(0, 48)])
    pltpu.sync_copy(ins_hbm.at[pl.ds(b0 * V, BL * V)], ids_v.at[pl.ds(48, 48)])
    pltpu.sync_copy(sub_hbm.at[pl.ds(gr0, NCELL)], ids_v.at[pl.ds(96, NCELL)])

    for k in range(_NINIT):
        idxv = tabi_v[pl.ds(_INITI0 + k * L, L)]
        valv = tabf_v[pl.ds(k * L, L)]
        plsc.store_scatter(alpha_v, [idxv], valv)
    plsc.store_scatter(compact_v, [tabi_v[pl.ds(_CINITI0, L)]],
                       tabf_v[pl.ds(_NINIT * L, L)])

    rows_lo = lax.iota(jnp.int32, L)

    def dp_chunk(i):
        off = pl.multiple_of(i * L, L)
        ai = tabi_v[pl.ds(_AIDX0 + off, L)]
        si = tabi_v[pl.ds(_SIDX0 + off, L)]
        ci = tabi_v[pl.ds(_CIDX0 + off, L)]
        insv = plsc.load_gather(dscores_v, [si])
        delv = plsc.load_gather(dscores_v, [si + NCELL])
        subv = plsc.load_gather(dscores_v, [si + 2 * NCELL])
        a_l = plsc.load_gather(alpha_v, [ai - 1])
        a_u = plsc.load_gather(alpha_v, [ai - W])
        a_d = plsc.load_gather(alpha_v, [ai - (W + 1)])
        x1 = insv + a_l
        x2 = delv + a_u
        x3 = subv + a_d
        m = jnp.maximum(x1, jnp.maximum(x2, x3))
        s = jnp.exp(x1 - m) + jnp.exp(x2 - m) + jnp.exp(x3 - m)
        r = m + _log1to4(s)
        plsc.store_scatter(alpha_v, [ai], r)
        plsc.store_scatter(compact_v, [ci], r)

    dp_done = 0
    for p in range(NP):
        for b in range(BL):
            wait(b, p)
            g = b * NP + p
            for s in range(3):
                cv = plsc.load_gather(
                    ids_v, [tabi_v[pl.ds(_EMAP0 + (g * 3 + s) * L, L)]])
                val = plsc.load_gather(blks[b][p % 2], [rows_lo, cv])
                dscores_v[pl.ds(s * NCELL + g * L, L)] = val
            if p + 2 < NP:
                fetch(b, p + 2)
        if _DP_HI[p] > dp_done:
            pl.loop(dp_done, _DP_HI[p])(dp_chunk)
            dp_done = _DP_HI[p]

    if dp_done < _NDP:
        pl.loop(dp_done, _NDP)(dp_chunk)

    for b in range(BL):
        pltpu.sync_copy(compact_v.at[pl.ds(b * TV, TV)],
                        out_hbm.at[pl.ds((b0 + b) * TV, TV)])


@jax.jit
def _edit_dist_sc(scores2d, del_flat, ins_flat, sub_flat):
    mesh = plsc.VectorSubcoreMesh(core_axis_name="c", subcore_axis_name="s",
                                  num_cores=NC, num_subcores=NS)
    fn = pl.kernel(
        _body,
        out_type=jax.ShapeDtypeStruct((B * TV,), jnp.float32),
        mesh=mesh,
        compiler_params=pltpu.CompilerParams(needs_layout_passes=False),
        scratch_types=[
            pltpu.VMEM((96 + NCELL,), jnp.int32),
            pltpu.VMEM((_TABI.size,), jnp.int32),
            pltpu.VMEM((_TABF.size,), jnp.float32),
            pltpu.VMEM((3 * NCELL,), jnp.float32),
            pltpu.VMEM((ASIZE,), jnp.float32),
            pltpu.VMEM((CSIZE,), jnp.float32),
            pltpu.VMEM((RCH, C), jnp.float32),
            pltpu.VMEM((RCH, C), jnp.float32),
            pltpu.VMEM((RCH, C), jnp.float32),
            pltpu.VMEM((RCH, C), jnp.float32),
            pltpu.SemaphoreType.DMA,
            pltpu.SemaphoreType.DMA,
            pltpu.SemaphoreType.DMA,
            pltpu.SemaphoreType.DMA,
        ],
    )
    return fn(scores2d, del_flat, ins_flat, sub_flat,
              jnp.asarray(_TABI), jnp.asarray(_TABF))


def kernel(all_deletion_ids, all_insertion_ids, all_subs_ids, action_scores):
    out = _edit_dist_sc(
        action_scores.reshape(B * T * V, C),
        all_deletion_ids.reshape(-1).astype(jnp.int32),
        all_insertion_ids.reshape(-1).astype(jnp.int32),
        all_subs_ids.reshape(-1).astype(jnp.int32),
    )
    return out.reshape(B, T, V)


# speedup vs baseline: 1.0876x; 1.0410x over previous
import jax
import jax.numpy as jnp
import numpy as np
from jax import lax
from jax.experimental import pallas as pl
from jax.experimental.pallas import tpu as pltpu
from jax.experimental.pallas import tpu_sc as plsc

B, T, V, C = 64, 24, 24, 1001
NC, NS, L = 2, 16, 16
NW = NC * NS
BL = B // NW

TV = T * V
NCELL = BL * TV
RCH = 16
NP = TV // RCH
W = V + 1
APB = 640
ADUM = 2 * APB
ASIZE = ADUM + 2 * L
CDUM = NCELL
CSIZE = CDUM + L
NEG = -1.0e30
_ABL_EXTRACT = True
_ABL_DP = True


def _build_tables():
    emap = np.zeros((NCELL // L, 3, L), np.int32)
    for g in range(NCELL // L):
        for s in range(3):
            for j in range(L):
                cc = g * L + j
                b, rem = divmod(cc, TV)
                t, v = divmod(rem, V)
                if s == 0:
                    emap[g, s, j] = 48 + b * V + v
                elif s == 1:
                    emap[g, s, j] = b * T + t
                else:
                    emap[g, s, j] = 96 + cc
    emap = emap.reshape(-1)

    aidx_rows, sidx_rows, cidx_rows, diag_of_chunk = [], [], [], []
    for d in range(1, T + V - 1):
        cells = [(b, t, d - t)
                 for b in range(BL)
                 for t in range(max(0, d - (V - 1)), min(T - 1, d) + 1)]
        for c0 in range(0, len(cells), L):
            chunk = cells[c0:c0 + L]
            ai = [ADUM + j for j in range(L)]
            si = [0] * L
            ci = [CDUM + j for j in range(L)]
            for j, (b, t, v) in enumerate(chunk):
                ai[j] = b * APB + (t + 1) * W + (v + 1)
                si[j] = b * TV + t * V + v
                ci[j] = si[j]
            aidx_rows.append(ai)
            sidx_rows.append(si)
            cidx_rows.append(ci)
            diag_of_chunk.append(d)
    aidx = np.array(aidx_rows, np.int32).reshape(-1)
    sidx = np.array(sidx_rows, np.int32).reshape(-1)
    cidx = np.array(cidx_rows, np.int32).reshape(-1)
    ndp = len(aidx_rows)

    maxrow = np.zeros(T + V - 1, np.int64)
    for d in range(1, T + V - 1):
        maxrow[d] = max(t * V + (d - t)
                        for t in range(max(0, d - (V - 1)), min(T - 1, d) + 1))
    prefmax = np.maximum.accumulate(maxrow)
    dp_hi = []
    for p in range(NP):
        bound = RCH * (p + 1)
        dmax = 0
        for d in range(1, T + V - 1):
            if prefmax[d] < bound:
                dmax = d
            else:
                break
        hi = sum(1 for dd in diag_of_chunk if dd <= dmax)
        dp_hi.append(hi)
    assert dp_hi[-1] <= ndp
    lo = 0
    for p in range(NP):
        for k in range(lo, dp_hi[p]):
            for cc in sidx[k * L:(k + 1) * L]:
                assert cc % TV < RCH * (p + 1), (p, k, cc)
        lo = dp_hi[p]

    init_entries = []
    for b in range(BL):
        for vv in range(W):
            init_entries.append((b * APB + vv, NEG))
        for tt in range(1, W):
            init_entries.append((b * APB + tt * W, NEG))
        init_entries.append((b * APB + W + 1, 0.0))
    for j in range(2 * L):
        init_entries.append((ADUM + j, NEG))
    pad = 0
    while len(init_entries) % L:
        init_entries.append((APB - 16 + pad, NEG))
        pad += 1
    init_idx = np.array([e[0] for e in init_entries], np.int32)
    init_val = np.array([e[1] for e in init_entries], np.float32)

    cinit_idx = np.array([0, TV] + [CDUM + j for j in range(L - 2)], np.int32)
    cinit_val = np.array([0.0, 0.0] + [NEG] * (L - 2), np.float32)

    tabi = np.concatenate([emap, aidx, sidx, cidx, init_idx, cinit_idx])
    tabf = np.concatenate([init_val, cinit_val])
    offs = {}
    o = 0
    for name, a in (("emap", emap), ("aidx", aidx), ("sidx", sidx),
                    ("cidx", cidx), ("init_idx", init_idx),
                    ("cinit_idx", cinit_idx)):
        offs[name] = o
        o += a.size
    return tabi, tabf, offs, ndp, init_idx.size // L, dp_hi


_TABI, _TABF, _OFFS, _NDP, _NINIT, _DP_HI = _build_tables()
_EMAP0 = _OFFS["emap"]
_AIDX0 = _OFFS["aidx"]
_SIDX0 = _OFFS["sidx"]
_CIDX0 = _OFFS["cidx"]
_INITI0 = _OFFS["init_idx"]
_CINITI0 = _OFFS["cinit_idx"]

_LN2 = 0.6931471805599453
_SQRT2 = 1.4142135623730951


def _log1to4(s):
    bits = plsc.bitcast(s, jnp.int32)
    e = (bits >> 23) - 127
    mant = plsc.bitcast((bits & 0x007FFFFF) | 0x3F800000, jnp.float32)
    big = mant > _SQRT2
    mant = jnp.where(big, mant * 0.5, mant)
    e = e + big.astype(jnp.int32)
    u = (mant - 1.0) / (mant + 1.0)
    u2 = u * u
    p = 2.0 * u * (1.0 + u2 * (1.0 / 3.0 + u2 * (0.2 + u2 * (1.0 / 7.0
                                                             + u2 * (1.0 / 9.0)))))
    return e.astype(jnp.float32) * _LN2 + p


def _body(scores_hbm, del_hbm, ins_hbm, sub_hbm, tabi_hbm, tabf_hbm,
          out_hbm,
          ids_v, tabi_v, tabf_v, dscores_v, alpha_v, compact_v,
          blk00, blk01, blk02, blk10, blk11, blk12,
          sem00, sem01, sem02, sem10, sem11, sem12):
    wid = lax.axis_index("s") * NC + lax.axis_index("c")
    b0 = wid * BL
    gr0 = b0 * TV

    NBUF = 3
    blks = ((blk00, blk01, blk02), (blk10, blk11, blk12))
    sems = ((sem00, sem01, sem02), (sem10, sem11, sem12))

    def fetch(b, q):
        pltpu.async_copy(
            scores_hbm.at[pl.ds(gr0 + b * TV + q * RCH, RCH), :],
            blks[b][q % NBUF], sems[b][q % NBUF])

    def wait(b, q):
        pltpu.make_async_copy(scores_hbm.at[pl.ds(0, RCH), :],
                              blks[b][q % NBUF], sems[b][q % NBUF]).wait()

    for q in range(NBUF):
        for b in range(BL):
            fetch(b, q)
    pltpu.sync_copy(tabi_hbm, tabi_v)
    pltpu.sync_copy(tabf_hbm, tabf_v)
    pltpu.sync_copy(del_hbm.at[pl.ds(b0 * T, BL * T)], ids_v.at[pl.ds(0, 48)])
    pltpu.sync_copy(ins_hbm.at[pl.ds(b0 * V, BL * V)], ids_v.at[pl.ds(48, 48)])
    pltpu.sync_copy(sub_hbm.at[pl.ds(gr0, NCELL)], ids_v.at[pl.ds(96, NCELL)])

    for k in range(_NINIT):
        idxv = tabi_v[pl.ds(_INITI0 + k * L, L)]
        valv = tabf_v[pl.ds(k * L, L)]
        plsc.store_scatter(alpha_v, [idxv], valv)
    plsc.store_scatter(compact_v, [tabi_v[pl.ds(_CINITI0, L)]],
                       tabf_v[pl.ds(_NINIT * L, L)])

    rows_lo = lax.iota(jnp.int32, L)

    def dp_chunk(i):
        off = pl.multiple_of(i * L, L)
        ai = tabi_v[pl.ds(_AIDX0 + off, L)]
        si = tabi_v[pl.ds(_SIDX0 + off, L)]
        ci = tabi_v[pl.ds(_CIDX0 + off, L)]
        insv = plsc.load_gather(dscores_v, [si])
        delv = plsc.load_gather(dscores_v, [si + NCELL])
        subv = plsc.load_gather(dscores_v, [si + 2 * NCELL])
        a_l = plsc.load_gather(alpha_v, [ai - 1])
        a_u = plsc.load_gather(alpha_v, [ai - W])
        a_d = plsc.load_gather(alpha_v, [ai - (W + 1)])
        x1 = insv + a_l
        x2 = delv + a_u
        x3 = subv + a_d
        m = jnp.maximum(x1, jnp.maximum(x2, x3))
        s = jnp.exp(x1 - m) + jnp.exp(x2 - m) + jnp.exp(x3 - m)
        r = m + _log1to4(s)
        plsc.store_scatter(alpha_v, [ai], r)
        plsc.store_scatter(compact_v, [ci], r)

    dp_done = 0
    for p in range(NP):
        for b in range(BL):
            wait(b, p)
            g = b * NP + p
            for s in range(3 if _ABL_EXTRACT else 0):
                cv = plsc.load_gather(
                    ids_v, [tabi_v[pl.ds(_EMAP0 + (g * 3 + s) * L, L)]])
                val = plsc.load_gather(blks[b][p % NBUF], [rows_lo, cv])
                dscores_v[pl.ds(s * NCELL + g * L, L)] = val
            if p + NBUF < NP:
                fetch(b, p + NBUF)
        if _ABL_DP and _DP_HI[p] > dp_done:
            pl.loop(dp_done, _DP_HI[p])(dp_chunk)
            dp_done = _DP_HI[p]

    if _ABL_DP and dp_done < _NDP:
        pl.loop(dp_done, _NDP)(dp_chunk)

    for b in range(BL):
        pltpu.sync_copy(compact_v.at[pl.ds(b * TV, TV)],
                        out_hbm.at[pl.ds((b0 + b) * TV, TV)])


@jax.jit
def _edit_dist_sc(scores2d, del_flat, ins_flat, sub_flat):
    mesh = plsc.VectorSubcoreMesh(core_axis_name="c", subcore_axis_name="s",
                                  num_cores=NC, num_subcores=NS)
    fn = pl.kernel(
        _body,
        out_type=jax.ShapeDtypeStruct((B * TV,), jnp.float32),
        mesh=mesh,
        compiler_params=pltpu.CompilerParams(needs_layout_passes=False),
        scratch_types=[
            pltpu.VMEM((96 + NCELL,), jnp.int32),
            pltpu.VMEM((_TABI.size,), jnp.int32),
            pltpu.VMEM((_TABF.size,), jnp.float32),
            pltpu.VMEM((3 * NCELL,), jnp.float32),
            pltpu.VMEM((ASIZE,), jnp.float32),
            pltpu.VMEM((CSIZE,), jnp.float32),
            pltpu.VMEM((RCH, C), jnp.float32),
            pltpu.VMEM((RCH, C), jnp.float32),
            pltpu.VMEM((RCH, C), jnp.float32),
            pltpu.VMEM((RCH, C), jnp.float32),
            pltpu.VMEM((RCH, C), jnp.float32),
            pltpu.VMEM((RCH, C), jnp.float32),
            pltpu.SemaphoreType.DMA,
            pltpu.SemaphoreType.DMA,
            pltpu.SemaphoreType.DMA,
            pltpu.SemaphoreType.DMA,
            pltpu.SemaphoreType.DMA,
            pltpu.SemaphoreType.DMA,
        ],
    )
    return fn(scores2d, del_flat, ins_flat, sub_flat,
              jnp.asarray(_TABI), jnp.asarray(_TABF))


def kernel(all_deletion_ids, all_insertion_ids, all_subs_ids, action_scores):
    out = _edit_dist_sc(
        action_scores.reshape(B * T * V, C),
        all_deletion_ids.reshape(-1).astype(jnp.int32),
        all_insertion_ids.reshape(-1).astype(jnp.int32),
        all_subs_ids.reshape(-1).astype(jnp.int32),
    )
    return out.reshape(B, T, V)
